# Initial kernel scaffold; baseline (speedup 1.0000x reference)
import jax
import jax.numpy as jnp
from jax.experimental import pallas as pl


def kernel(nodes, edges, senders, receivers, n_node, n_edge, params):
    raise NotImplementedError("write your pallas kernel here")


# speedup vs baseline: 1.4834x; 1.4834x over previous
import functools

import jax
import jax.numpy as jnp
from jax import lax
from jax.experimental import pallas as pl
from jax.experimental.pallas import tpu as pltpu
from jax.experimental.pallas import tpu_sc as plsc

N = 10000
E = 160000
D = 256
NUM_OUT = 128

CH = 128
NCH = E // CH
R_E = 2000
R_N = 1000
NB_E = E // R_E
NB_N = N // R_N
NPAD = 10240
NBLK = 128
WIN = 640
NWIN = E // WIN
NGB = NPAD // NBLK


def _ln_relu(h, g, beta):
    mu = jnp.mean(h, axis=-1, keepdims=True)
    var = jnp.mean((h - mu) ** 2, axis=-1, keepdims=True)
    hn = (h - mu) * lax.rsqrt(var + 1e-5)
    return jnp.maximum(hn * g + beta, 0.0)


def _init_body(x_ref, wemb_ref, bemb_ref, wp1_ref, wp2_ref, wp3_ref,
               p1_ref, p2_ref, p3_ref):
    h = jnp.dot(x_ref[...], wemb_ref[...], preferred_element_type=jnp.float32)
    h = h + bemb_ref[...]
    p1_ref[...] = jnp.dot(h, wp1_ref[...], preferred_element_type=jnp.float32)
    p2_ref[...] = jnp.dot(h, wp2_ref[...], preferred_element_type=jnp.float32)
    p3_ref[...] = jnp.dot(h, wp3_ref[...], preferred_element_type=jnp.float32)


def _node_init(x, wemb, bemb, wp1, wp2, wp3):
    blk = functools.partial(pl.BlockSpec, index_map=lambda i: (0, 0))
    return pl.pallas_call(
        _init_body,
        grid=(NB_N,),
        in_specs=[
            pl.BlockSpec((R_N, 9), lambda i: (i, 0)),
            blk((9, D)), blk((1, D)), blk((D, D)), blk((D, D)), blk((D, D)),
        ],
        out_specs=[pl.BlockSpec((R_N, D), lambda i: (i, 0))] * 3,
        out_shape=[jax.ShapeDtypeStruct((N, D), jnp.float32)] * 3,
    )(x, wemb, bemb, wp1, wp2, wp3)


def _edge_body(x_ref, cs_ref, cr_ref, grow_ref, w_ref, b_ref, g_ref, beta_ref,
               out_ref, psum_ref):
    h = jnp.dot(x_ref[...], w_ref[...], preferred_element_type=jnp.float32)
    h = h + cs_ref[...] + cr_ref[...] + grow_ref[...] + b_ref[...]
    y = _ln_relu(h, g_ref[...], beta_ref[...])
    out_ref[...] = y
    psum_ref[0] = jnp.sum(y, axis=0, keepdims=True)


def _edge_update(x, cs, cr, grow, w, b, g, beta):
    in_dim = x.shape[1]
    blk = functools.partial(pl.BlockSpec, index_map=lambda i: (0, 0))
    return pl.pallas_call(
        _edge_body,
        grid=(NB_E,),
        in_specs=[
            pl.BlockSpec((R_E, in_dim), lambda i: (i, 0)),
            pl.BlockSpec((R_E, D), lambda i: (i, 0)),
            pl.BlockSpec((R_E, D), lambda i: (i, 0)),
            blk((1, D)), blk((in_dim, D)), blk((1, D)), blk((1, D)), blk((1, D)),
        ],
        out_specs=[pl.BlockSpec((R_E, D), lambda i: (i, 0)),
                   pl.BlockSpec((1, 1, D), lambda i: (i, 0, 0))],
        out_shape=[jax.ShapeDtypeStruct((E, D), jnp.float32),
                   jax.ShapeDtypeStruct((NB_E, 1, D), jnp.float32)],
    )(x, cs, cr, grow, w, b, g, beta)


def _node_body_proj(p1_ref, zs_ref, zr_ref, grow_ref, w2_ref, w3_ref,
                    b_ref, g_ref, beta_ref, wp1_ref, wp2_ref, wp3_ref,
                    psum_ref, q1_ref, q2_ref, q3_ref):
    h = p1_ref[...]
    h = h + jnp.dot(zs_ref[...], w2_ref[...], preferred_element_type=jnp.float32)
    h = h + jnp.dot(zr_ref[...], w3_ref[...], preferred_element_type=jnp.float32)
    h = h + grow_ref[...] + b_ref[...]
    y = _ln_relu(h, g_ref[...], beta_ref[...])
    psum_ref[0] = jnp.sum(y, axis=0, keepdims=True)
    q1_ref[...] = jnp.dot(y, wp1_ref[...], preferred_element_type=jnp.float32)
    q2_ref[...] = jnp.dot(y, wp2_ref[...], preferred_element_type=jnp.float32)
    q3_ref[...] = jnp.dot(y, wp3_ref[...], preferred_element_type=jnp.float32)


def _node_body_last(p1_ref, zs_ref, zr_ref, grow_ref, w2_ref, w3_ref,
                    b_ref, g_ref, beta_ref, psum_ref):
    h = p1_ref[...]
    h = h + jnp.dot(zs_ref[...], w2_ref[...], preferred_element_type=jnp.float32)
    h = h + jnp.dot(zr_ref[...], w3_ref[...], preferred_element_type=jnp.float32)
    h = h + grow_ref[...] + b_ref[...]
    y = _ln_relu(h, g_ref[...], beta_ref[...])
    psum_ref[0] = jnp.sum(y, axis=0, keepdims=True)


def _node_update(p1, zs, zr, grow, w2, w3, b, g, beta, proj_w):
    blk = functools.partial(pl.BlockSpec, index_map=lambda i: (0, 0))
    zspec = pl.BlockSpec((R_N, D), lambda i: (i, 0))
    in_specs = [
        pl.BlockSpec((R_N, D), lambda i: (i, 0)),
        zspec, zspec,
        blk((1, D)), blk((D, D)), blk((D, D)), blk((1, D)), blk((1, D)),
        blk((1, D)),
    ]
    if proj_w is None:
        return pl.pallas_call(
            _node_body_last,
            grid=(NB_N,),
            in_specs=in_specs,
            out_specs=[pl.BlockSpec((1, 1, D), lambda i: (i, 0, 0))],
            out_shape=[jax.ShapeDtypeStruct((NB_N, 1, D), jnp.float32)],
        )(p1, zs, zr, grow, w2, w3, b, g, beta)
    return pl.pallas_call(
        _node_body_proj,
        grid=(NB_N,),
        in_specs=in_specs + [blk((D, D))] * 3,
        out_specs=[pl.BlockSpec((1, 1, D), lambda i: (i, 0, 0))] +
                  [pl.BlockSpec((R_N, D), lambda i: (i, 0))] * 3,
        out_shape=[jax.ShapeDtypeStruct((NB_N, 1, D), jnp.float32)] +
                  [jax.ShapeDtypeStruct((N, D), jnp.float32)] * 3,
    )(p1, zs, zr, grow, w2, w3, b, g, beta, *proj_w)


def _glob_body_mid(npsum_ref, epsum_ref, gs_ref, wn_ref, we_ref, wg_ref,
                   b_ref, g_ref, beta_ref, wge_ref, wgn_ref,
                   gs2_ref, ge_ref, gn_ref):
    na = jnp.sum(npsum_ref[...], axis=(0, 1))[None]
    ea = jnp.sum(epsum_ref[...], axis=(0, 1))[None]
    h = jnp.dot(na, wn_ref[...], preferred_element_type=jnp.float32)
    h = h + jnp.dot(ea, we_ref[...], preferred_element_type=jnp.float32)
    h = h + jnp.dot(gs_ref[...], wg_ref[...], preferred_element_type=jnp.float32)
    h = h + b_ref[...]
    y = _ln_relu(h, g_ref[...], beta_ref[...])
    gs2_ref[...] = y
    ge_ref[...] = jnp.dot(y, wge_ref[...], preferred_element_type=jnp.float32)
    gn_ref[...] = jnp.dot(y, wgn_ref[...], preferred_element_type=jnp.float32)


def _glob_body_last(npsum_ref, epsum_ref, gs_ref, wn_ref, we_ref, wg_ref,
                    b_ref, g_ref, beta_ref, wdec_ref, bdec_ref, out_ref):
    na = jnp.sum(npsum_ref[...], axis=(0, 1))[None]
    ea = jnp.sum(epsum_ref[...], axis=(0, 1))[None]
    h = jnp.dot(na, wn_ref[...], preferred_element_type=jnp.float32)
    h = h + jnp.dot(ea, we_ref[...], preferred_element_type=jnp.float32)
    h = h + jnp.dot(gs_ref[...], wg_ref[...], preferred_element_type=jnp.float32)
    h = h + b_ref[...]
    y = _ln_relu(h, g_ref[...], beta_ref[...])
    out_ref[...] = jnp.dot(y, wdec_ref[...], preferred_element_type=jnp.float32)
    out_ref[...] += bdec_ref[...]


def _glob_update(npsum, epsum, gs, wn, we, wg, b, g, beta, nxt):
    gd = gs.shape[1]
    specs = [
        pl.BlockSpec((NB_N, 1, D), lambda: (0, 0, 0)),
        pl.BlockSpec((NB_E, 1, D), lambda: (0, 0, 0)),
        pl.BlockSpec((1, gd), lambda: (0, 0)),
        pl.BlockSpec((D, D), lambda: (0, 0)),
        pl.BlockSpec((D, D), lambda: (0, 0)),
        pl.BlockSpec((gd, D), lambda: (0, 0)),
        pl.BlockSpec((1, D), lambda: (0, 0)),
        pl.BlockSpec((1, D), lambda: (0, 0)),
        pl.BlockSpec((1, D), lambda: (0, 0)),
    ]
    if nxt[0] is None:
        wdec, bdec = nxt[1]
        return pl.pallas_call(
            _glob_body_last,
            in_specs=specs + [pl.BlockSpec((D, NUM_OUT), lambda: (0, 0)),
                              pl.BlockSpec((1, NUM_OUT), lambda: (0, 0))],
            out_specs=pl.BlockSpec((1, NUM_OUT), lambda: (0, 0)),
            out_shape=jax.ShapeDtypeStruct((1, NUM_OUT), jnp.float32),
        )(npsum, epsum, gs, wn, we, wg, b, g, beta, wdec, bdec)
    wge, wgn = nxt
    return pl.pallas_call(
        _glob_body_mid,
        in_specs=specs + [pl.BlockSpec((D, D), lambda: (0, 0))] * 2,
        out_specs=[pl.BlockSpec((1, D), lambda: (0, 0))] * 3,
        out_shape=[jax.ShapeDtypeStruct((1, D), jnp.float32)] * 3,
    )(npsum, epsum, gs, wn, we, wg, b, g, beta, wge, wgn)


@functools.cache
def _sc_mesh():
    return plsc.VectorSubcoreMesh(core_axis_name="c", subcore_axis_name="s")


def _gather_kernel(ps_hbm, pr_hbm, sidx_hbm, ridx_hbm, outs_hbm, outr_hbm,
                   ib_s, ib_r, buf_s, buf_r, sem, sem2):
    c = lax.axis_index("c")
    s = lax.axis_index("s")
    w = s * 2 + c
    nch = 39 + (w < NCH - 39 * 32).astype(jnp.int32)

    def body(j, _):
        base = (w + 32 * j) * CH
        pltpu.sync_copy(sidx_hbm.at[pl.ds(base, CH)], ib_s)
        pltpu.sync_copy(ridx_hbm.at[pl.ds(base, CH)], ib_r)
        cpa = pltpu.async_copy(ps_hbm.at[ib_s], buf_s, sem)
        cpb = pltpu.async_copy(pr_hbm.at[ib_r], buf_r, sem2)
        cpa.wait()
        pltpu.sync_copy(buf_s, outs_hbm.at[pl.ds(base, CH)])
        cpb.wait()
        pltpu.sync_copy(buf_r, outr_hbm.at[pl.ds(base, CH)])
        return 0

    lax.fori_loop(0, nch, body, 0)


def _sc_gather(ps, pr, senders, receivers):
    f = pl.kernel(
        _gather_kernel,
        mesh=_sc_mesh(),
        out_type=[jax.ShapeDtypeStruct((E, D), jnp.float32)] * 2,
        scratch_types=[
            pltpu.VMEM((CH,), jnp.int32),
            pltpu.VMEM((CH,), jnp.int32),
            pltpu.VMEM((CH, D), jnp.float32),
            pltpu.VMEM((CH, D), jnp.float32),
            pltpu.SemaphoreType.DMA,
            pltpu.SemaphoreType.DMA,
        ],
    )
    return f(ps, pr, senders, receivers)


def _permute_kernel(edges_hbm, perm_hbm, out_hbm, ib, buf, sem):
    c = lax.axis_index("c")
    s = lax.axis_index("s")
    w = s * 2 + c
    nch = 39 + (w < NCH - 39 * 32).astype(jnp.int32)

    def body(j, _):
        base = (w + 32 * j) * CH
        for a in range(2):
            pltpu.sync_copy(perm_hbm.at[a, pl.ds(base, CH)], ib)
            pltpu.async_copy(edges_hbm.at[ib], buf, sem).wait()
            pltpu.sync_copy(buf, out_hbm.at[a, pl.ds(base, CH)])
        return 0

    lax.fori_loop(0, nch, body, 0)


def _sc_permute(edges, perm2):
    f = pl.kernel(
        _permute_kernel,
        mesh=_sc_mesh(),
        out_type=jax.ShapeDtypeStruct((2, E, D), jnp.float32),
        scratch_types=[
            pltpu.VMEM((CH,), jnp.int32),
            pltpu.VMEM((CH, D), jnp.float32),
            pltpu.SemaphoreType.DMA,
        ],
    )
    return f(edges, perm2)


def _segred_body(offs_ref, sidx_ref, rows_ref, out_ref, ibuf, rbuf, sem1, sem2):
    a = pl.program_id(0)
    nb = pl.program_id(1)
    s = offs_ref[a, nb]
    e = offs_ref[a, nb + 1]
    k0 = s // WIN
    k1 = (e + WIN - 1) // WIN

    def body(k, acc):
        cp1 = pltpu.make_async_copy(sidx_ref.at[a, pl.ds(k, 1)], ibuf, sem1)
        cp2 = pltpu.make_async_copy(rows_ref.at[a, pl.ds(k * WIN, WIN)], rbuf, sem2)
        cp1.start()
        cp2.start()
        cp1.wait()
        cp2.wait()
        pos = k * WIN + lax.broadcasted_iota(jnp.int32, (1, WIN), 1)
        valid = (pos >= s) & (pos < e)
        lidx = ibuf[...] - nb * NBLK
        niota = lax.broadcasted_iota(jnp.int32, (NBLK, WIN), 0)
        oh = jnp.where((niota == lidx) & valid, 1.0, 0.0).astype(jnp.float32)
        return acc + jnp.dot(oh, rbuf[...], preferred_element_type=jnp.float32)

    acc = lax.fori_loop(k0, k1, body, jnp.zeros((NBLK, D), jnp.float32))
    out_ref[0] = acc


def _seg_reduce(offs, sidx3, sorted_rows):
    return pl.pallas_call(
        _segred_body,
        grid=(2, NGB),
        in_specs=[
            pl.BlockSpec(memory_space=pltpu.MemorySpace.SMEM),
            pl.BlockSpec(memory_space=pl.ANY),
            pl.BlockSpec(memory_space=pl.ANY),
        ],
        out_specs=pl.BlockSpec((1, NBLK, D), lambda a, nb: (a, nb, 0)),
        out_shape=jax.ShapeDtypeStruct((2, NPAD, D), jnp.float32),
        scratch_shapes=[
            pltpu.VMEM((1, WIN), jnp.int32),
            pltpu.VMEM((WIN, D), jnp.float32),
            pltpu.SemaphoreType.DMA,
            pltpu.SemaphoreType.DMA,
        ],
    )(offs, sidx3, sorted_rows)


def _row(v):
    return v.reshape(1, -1)


def kernel(nodes, edges, senders, receivers, n_node, n_edge, params):
    lp = params["layers"]

    def eW(st):
        w = lp[st]["edge"]["W"]
        return w[0:D], w[D:2 * D], w[2 * D:3 * D], w[3 * D:]

    def nW(st):
        w = lp[st]["node"]["W"]
        return w[0:D], w[D:2 * D], w[2 * D:3 * D], w[3 * D:]

    def gW(st):
        w = lp[st]["global"]["W"]
        return w[0:D], w[D:2 * D], w[2 * D:]

    we_e0, we_s0, we_r0, _ = eW(0)
    wfold = params["edge_emb"]["W"] @ we_e0
    bfold = _row(params["edge_emb"]["b"] @ we_e0 + lp[0]["edge"]["b"])

    wn1_0, _, _, _ = nW(0)
    p1, ps, pr = _node_init(
        nodes, params["node_emb"]["W"], _row(params["node_emb"]["b"]),
        wn1_0, we_s0, we_r0)

    perms, sidxs, offss = [], [], []
    for ix in (senders, receivers):
        pm = jnp.argsort(ix)
        si = ix[pm]
        offss.append(jnp.searchsorted(si, jnp.arange(NGB + 1, dtype=jnp.int32) * NBLK).astype(jnp.int32))
        perms.append(pm.astype(jnp.int32))
        sidxs.append(si)
    perm2 = jnp.stack(perms)
    offs = jnp.stack(offss)
    npad_idx = (-E) % WIN
    sidx3 = jnp.stack([
        jnp.concatenate([si, jnp.zeros((npad_idx,), jnp.int32)]).reshape(-1, WIN)
        for si in sidxs])
    zero_row = jnp.zeros((1, D), jnp.float32)
    ge_row = zero_row
    gn_row = zero_row
    gs = jnp.zeros((1, NUM_OUT), jnp.float32)
    edges_x = edges

    out = None
    for st in range(5):
        ep = lp[st]["edge"]
        npp = lp[st]["node"]
        gp = lp[st]["global"]

        cs, cr = _sc_gather(ps, pr, senders, receivers)

        w_e = wfold if st == 0 else eW(st)[0]
        b_e = bfold if st == 0 else _row(ep["b"])
        edges_x, epsum = _edge_update(
            edges_x, cs, cr, ge_row, w_e, b_e, _row(ep["g"]), _row(ep["beta"]))

        sorted_rows = _sc_permute(edges_x, perm2)
        z = _seg_reduce(offs, sidx3, sorted_rows)
        zs, zr = z[0], z[1]

        _, wn2, wn3, _ = nW(st)
        if st < 4:
            wn1_n, _, _, _ = nW(st + 1)
            _, we_s_n, we_r_n, _ = eW(st + 1)
            npsum, p1, ps, pr = _node_update(
                p1, zs, zr, gn_row, wn2, wn3, _row(npp["b"]), _row(npp["g"]),
                _row(npp["beta"]), (wn1_n, we_s_n, we_r_n))
        else:
            (npsum,) = _node_update(
                p1, zs, zr, gn_row, wn2, wn3, _row(npp["b"]), _row(npp["g"]),
                _row(npp["beta"]), None)

        wg_n, wg_e, wg_g = gW(st)
        if st < 4:
            we_g_n = eW(st + 1)[3]
            wn_g_n = nW(st + 1)[3]
            gs, ge_row, gn_row = _glob_update(
                npsum, epsum, gs, wg_n, wg_e, wg_g, _row(gp["b"]),
                _row(gp["g"]), _row(gp["beta"]), (we_g_n, wn_g_n))
        else:
            out = _glob_update(
                npsum, epsum, gs, wg_n, wg_e, wg_g, _row(gp["b"]),
                _row(gp["g"]), _row(gp["beta"]),
                (None, (params["dec"]["W"], _row(params["dec"]["b"]))))

    return out
